# baseline (speedup 1.0000x reference)
import jax
import jax.numpy as jnp
from jax import lax
from jax.experimental import pallas as pl
from jax.experimental.pallas import tpu as pltpu
from jax.experimental.pallas import tpu_sc as plsc

_NUM_FIELDS = 26
_VOCAB = 100000
_DIM = 32
_BATCH = 16384

_NC = 2
_NS = 16
_NW = _NC * _NS
_HALF = _BATCH // 2
_NVEC = _HALF // 16
_VA = 49920
_VB = _VOCAB - _VA


def _sc_body(
    xt_hbm, tt_hbm, out_hbm, tabA_v, tabB_v, idx_v, ob_v, xsh, tsem, isem, wsem, xsem
):
    sid = lax.axis_index("s")
    w = sid * _NC + lax.axis_index("c")

    @pl.when(sid == 0)
    def _():
        pltpu.async_copy(xt_hbm.at[0], xsh.at[0], xsem).wait()

    def tab_half(f, a):
        if a == 0:
            return pltpu.make_async_copy(
                tt_hbm.at[f, w, pl.ds(0, _VA)], tabA_v, tsem
            )
        return pltpu.make_async_copy(
            tt_hbm.at[f, w, pl.ds(_VA, _VB)], tabB_v, tsem
        )

    tab_half(0, 0).start()
    tab_half(0, 1).start()

    def field(f, carry):
        plsc.subcore_barrier()
        p = lax.rem(f, 2)

        @pl.when(jnp.logical_and(sid == 0, f + 1 < _NUM_FIELDS))
        def _():
            pltpu.async_copy(xt_hbm.at[f + 1], xsh.at[1 - p], xsem)

        i0 = pltpu.async_copy(xsh.at[p, pl.ds(0, _HALF)], idx_v.at[0], isem)
        i1 = pltpu.async_copy(xsh.at[p, pl.ds(_HALF, _HALF)], idx_v.at[1], isem)
        tab_half(f, 0).wait()

        r = f * _DIM + w

        def run_pass(h, a):
            def g(k, c):
                ss = [pl.multiple_of(k * 128 + u * 16, 16) for u in range(8)]
                ivs = [idx_v[h, pl.ds(s, 16)] for s in ss]
                if a == 0:
                    ms = [iv < _VA for iv in ivs]
                    gs = [iv for iv in ivs]
                    tv = tabA_v
                else:
                    ms = [iv >= _VA for iv in ivs]
                    gs = [iv - _VA for iv in ivs]
                    tv = tabB_v
                vals = [
                    plsc.load_gather(tv, [g_], mask=m)
                    for g_, m in zip(gs, ms)
                ]
                for s, m, v in zip(ss, ms, vals):
                    if a == 0:
                        ob_v[pl.ds(s, 16)] = jnp.where(m, v, 0.0)
                    else:
                        ob_v[pl.ds(s, 16)] = jnp.where(m, v, ob_v[pl.ds(s, 16)])
                return c

            lax.fori_loop(0, _NVEC // 8, g, 0)

        for h in range(2):
            (i0 if h == 0 else i1).wait()

            @pl.when((f + h) >= 1)
            def _():
                rp = r if h == 1 else r - _DIM
                hp = 1 - h
                pltpu.make_async_copy(
                    ob_v, out_hbm.at[rp, pl.ds(hp * _HALF, _HALF)], wsem
                ).wait()

            run_pass(h, 0)
            if h == 0:
                tab_half(f, 1).wait()
            else:
                @pl.when(f + 1 < _NUM_FIELDS)
                def _():
                    tab_half(f + 1, 0).start()

            run_pass(h, 1)
            pltpu.async_copy(
                ob_v, out_hbm.at[r, pl.ds(h * _HALF, _HALF)], wsem
            )

        @pl.when(f + 1 < _NUM_FIELDS)
        def _():
            tab_half(f + 1, 1).start()

        @pl.when(jnp.logical_and(sid == 0, f + 1 < _NUM_FIELDS))
        def _():
            pltpu.make_async_copy(xt_hbm.at[f + 1], xsh.at[1 - p], xsem).wait()
        return carry

    lax.fori_loop(0, _NUM_FIELDS, field, 0)

    r_last = (_NUM_FIELDS - 1) * _DIM + w
    pltpu.make_async_copy(
        ob_v, out_hbm.at[r_last, pl.ds(_HALF, _HALF)], wsem
    ).wait()


@jax.jit
def _binned_embed(x_binned, tables):
    xt = x_binned.T
    tt = jnp.transpose(tables, (0, 2, 1))
    mesh = plsc.VectorSubcoreMesh(core_axis_name="c", subcore_axis_name="s")
    f = pl.kernel(
        _sc_body,
        out_type=jax.ShapeDtypeStruct((_NUM_FIELDS * _DIM, _BATCH), jnp.float32),
        mesh=mesh,
        scratch_types=[
            pltpu.VMEM((_VA,), jnp.float32),
            pltpu.VMEM((_VB,), jnp.float32),
            pltpu.VMEM((2, _HALF), jnp.int32),
            pltpu.VMEM((_HALF,), jnp.float32),
            pltpu.VMEM_SHARED((2, _BATCH), jnp.int32),
            pltpu.SemaphoreType.DMA,
            pltpu.SemaphoreType.DMA,
            pltpu.SemaphoreType.DMA,
            pltpu.SemaphoreType.DMA,
        ],
        compiler_params=pltpu.CompilerParams(
            use_tc_tiling_on_sc=True, needs_layout_passes=False
        ),
    )
    return f(xt, tt).T


def kernel(x_binned, tables):
    return _binned_embed(x_binned, tables)


# speedup vs baseline: 1.0394x; 1.0394x over previous
import jax
import jax.numpy as jnp
from jax import lax
from jax.experimental import pallas as pl
from jax.experimental.pallas import tpu as pltpu
from jax.experimental.pallas import tpu_sc as plsc

_NUM_FIELDS = 26
_VOCAB = 100000
_DIM = 32
_BATCH = 16384

_NC = 2
_NS = 16
_NW = _NC * _NS
_HALF = _BATCH // 2
_NVEC = _HALF // 16


def _sc_body(xt_hbm, tt_hbm, out_hbm, tab_v, idx_v, ob_v, xsh, tsem, isem, wsem, xsem):
    sid = lax.axis_index("s")
    w = sid * _NC + lax.axis_index("c")

    @pl.when(sid == 0)
    def _():
        pltpu.async_copy(xt_hbm.at[0], xsh.at[0], xsem).wait()

    def field(f, carry):
        plsc.subcore_barrier()
        p = lax.rem(f, 2)

        @pl.when(jnp.logical_and(sid == 0, f + 1 < _NUM_FIELDS))
        def _():
            pltpu.async_copy(xt_hbm.at[f + 1], xsh.at[1 - p], xsem)

        td = pltpu.async_copy(tt_hbm.at[f, w], tab_v, tsem)
        i0 = pltpu.async_copy(xsh.at[p, pl.ds(0, _HALF)], idx_v.at[0], isem)
        i1 = pltpu.async_copy(xsh.at[p, pl.ds(_HALF, _HALF)], idx_v.at[1], isem)
        td.wait()

        r = f * _DIM + w
        for h in range(2):
            (i0 if h == 0 else i1).wait()

            @pl.when((f + h) >= 1)
            def _():
                rp = r if h == 1 else r - _DIM
                hp = 1 - h
                pltpu.make_async_copy(
                    ob_v, out_hbm.at[rp, pl.ds(hp * _HALF, _HALF)], wsem
                ).wait()

            def g(k, c):
                ss = [pl.multiple_of(k * 128 + u * 16, 16) for u in range(8)]
                ivs = [idx_v[h, pl.ds(s, 16)] for s in ss]
                vals = [plsc.load_gather(tab_v, [iv]) for iv in ivs]
                for s, v in zip(ss, vals):
                    ob_v[pl.ds(s, 16)] = v
                return c

            lax.fori_loop(0, _NVEC // 8, g, 0)

            pltpu.async_copy(
                ob_v, out_hbm.at[r, pl.ds(h * _HALF, _HALF)], wsem
            )

        @pl.when(jnp.logical_and(sid == 0, f + 1 < _NUM_FIELDS))
        def _():
            pltpu.make_async_copy(xt_hbm.at[f + 1], xsh.at[1 - p], xsem).wait()
        return carry

    lax.fori_loop(0, _NUM_FIELDS, field, 0)

    r_last = (_NUM_FIELDS - 1) * _DIM + w
    pltpu.make_async_copy(
        ob_v, out_hbm.at[r_last, pl.ds(_HALF, _HALF)], wsem
    ).wait()


@jax.jit
def _binned_embed(x_binned, tables):
    xt = x_binned.T
    tt = jnp.transpose(tables, (0, 2, 1))
    mesh = plsc.VectorSubcoreMesh(core_axis_name="c", subcore_axis_name="s")
    f = pl.kernel(
        _sc_body,
        out_type=jax.ShapeDtypeStruct((_NUM_FIELDS * _DIM, _BATCH), jnp.float32),
        mesh=mesh,
        scratch_types=[
            pltpu.VMEM((_VOCAB,), jnp.float32),
            pltpu.VMEM((2, _HALF), jnp.int32),
            pltpu.VMEM((_HALF,), jnp.float32),
            pltpu.VMEM_SHARED((2, _BATCH), jnp.int32),
            pltpu.SemaphoreType.DMA,
            pltpu.SemaphoreType.DMA,
            pltpu.SemaphoreType.DMA,
            pltpu.SemaphoreType.DMA,
        ],
        compiler_params=pltpu.CompilerParams(
            use_tc_tiling_on_sc=True, needs_layout_passes=False
        ),
    )
    return f(xt, tt).T


def kernel(x_binned, tables):
    return _binned_embed(x_binned, tables)
